# baseline (speedup 1.0000x reference)
import jax
import jax.numpy as jnp
from jax import lax
from jax.experimental import pallas as pl
from jax.experimental.pallas import tpu as pltpu

_N = 100000
_D = 128
_CHUNK = 2000
_NCH = _N // _CHUNK


def _att_kernel(w0_ref, w1_ref, feat, out, ib0, ib1, ob0, ob1,
                si0, si1, so0, so1):
    w0 = w0_ref[...]
    w1 = w1_ref[...]

    def rd(buf, sem, row):
        return pltpu.make_async_copy(feat.at[pl.ds(row, _CHUNK)], buf, sem)

    def wr(buf, sem, row):
        return pltpu.make_async_copy(buf, out.at[pl.ds(row, _CHUNK)], sem)

    rd(ib0, si0, 0).start()
    rd(ib1, si1, _CHUNK).start()

    def pair(k, carry):
        for ib, ob, si, so, par in ((ib0, ob0, si0, so0, 0),
                                    (ib1, ob1, si1, so1, 1)):
            ch = 2 * k + par
            row = pl.multiple_of(ch * _CHUNK, 8)
            rd(ib, si, row).wait()

            @pl.when(ch >= 2)
            def _():
                prow = pl.multiple_of((ch - 2) * _CHUNK, 8)
                wr(ob, so, prow).wait()

            ob[...] = jnp.maximum(ib[...] * w0, 0.0) * w1
            wr(ob, so, row).start()

            @pl.when(ch + 2 < _NCH)
            def _():
                nrow = pl.multiple_of((ch + 2) * _CHUNK, 8)
                rd(ib, si, nrow).start()
        return carry

    lax.fori_loop(0, _NCH // 2, pair, 0)
    wr(ob0, so0, (_NCH - 2) * _CHUNK).wait()
    wr(ob1, so1, (_NCH - 1) * _CHUNK).wait()


def kernel(features, w0, w1):
    n, d = features.shape
    return pl.pallas_call(
        _att_kernel,
        in_specs=[
            pl.BlockSpec(memory_space=pltpu.VMEM),
            pl.BlockSpec(memory_space=pltpu.VMEM),
            pl.BlockSpec(memory_space=pl.ANY),
        ],
        out_specs=pl.BlockSpec(memory_space=pl.ANY),
        out_shape=jax.ShapeDtypeStruct((n, d), features.dtype),
        scratch_shapes=[
            pltpu.VMEM((_CHUNK, _D), jnp.float32),
            pltpu.VMEM((_CHUNK, _D), jnp.float32),
            pltpu.VMEM((_CHUNK, _D), jnp.float32),
            pltpu.VMEM((_CHUNK, _D), jnp.float32),
            pltpu.SemaphoreType.DMA,
            pltpu.SemaphoreType.DMA,
            pltpu.SemaphoreType.DMA,
            pltpu.SemaphoreType.DMA,
        ],
    )(w0.reshape(1, d), w1.reshape(1, d), features)
